# baseline (device time: 29804 ns/iter reference)
import jax
import jax.numpy as jnp
from jax import lax
from jax.experimental import pallas as pl
from jax.experimental.pallas import tpu as pltpu

N_DEV = 32
N_STEPS = 5


def kernel(x, W1, W2):
    m, k = x.shape
    _, h = W1.shape
    _, n = W2.shape

    def body(x_ref, w1_ref, w2_ref, out_ref,
             acc_ref, send_ref, recv_ref, send_sems, recv_sems):
        my = lax.axis_index("i")

        barrier_sem = pltpu.get_barrier_semaphore()
        for s in range(N_STEPS):
            partner = my ^ (1 << s)
            pl.semaphore_signal(
                barrier_sem, inc=1,
                device_id=(partner,), device_id_type=pl.DeviceIdType.MESH,
            )
        pl.semaphore_wait(barrier_sem, N_STEPS)

        xb = x_ref[...].astype(jnp.bfloat16)
        w1b = w1_ref[...].astype(jnp.bfloat16)
        w2b = w2_ref[...].astype(jnp.bfloat16)
        hid = jnp.dot(xb, w1b, preferred_element_type=jnp.float32)
        hid = jnp.maximum(hid, 0.0).astype(jnp.bfloat16)
        acc_ref[...] = jnp.dot(hid, w2b, preferred_element_type=jnp.float32)

        for s in range(N_STEPS):
            partner = my ^ (1 << s)
            send_ref[...] = acc_ref[...].astype(jnp.bfloat16)
            rdma = pltpu.make_async_remote_copy(
                src_ref=send_ref,
                dst_ref=recv_ref.at[s],
                send_sem=send_sems.at[s],
                recv_sem=recv_sems.at[s],
                device_id=(partner,),
                device_id_type=pl.DeviceIdType.MESH,
            )
            rdma.start()
            rdma.wait()
            acc_ref[...] = acc_ref[...] + recv_ref[s].astype(jnp.float32)

        out_ref[...] = acc_ref[...]

    return pl.pallas_call(
        body,
        out_shape=jax.ShapeDtypeStruct((m, n), jnp.float32),
        in_specs=[pl.BlockSpec(memory_space=pltpu.VMEM)] * 3,
        out_specs=pl.BlockSpec(memory_space=pltpu.VMEM),
        scratch_shapes=[
            pltpu.VMEM((m, n), jnp.float32),
            pltpu.VMEM((m, n), jnp.bfloat16),
            pltpu.VMEM((N_STEPS, m, n), jnp.bfloat16),
            pltpu.SemaphoreType.DMA((N_STEPS,)),
            pltpu.SemaphoreType.DMA((N_STEPS,)),
        ],
        compiler_params=pltpu.CompilerParams(collective_id=0),
    )(x, W1, W2)


# device time: 28354 ns/iter; 1.0511x vs baseline; 1.0511x over previous
import jax
import jax.numpy as jnp
from jax import lax
from jax.experimental import pallas as pl
from jax.experimental.pallas import tpu as pltpu

N_DEV = 32
N_STEPS = 5


def _partners(my):
    z = my // 8
    p = my % 8
    y = p // 2
    x = (p + y) % 2

    def to_idx(xx, yy, zz):
        return 8 * zz + 2 * yy + (xx + yy) % 2

    return [
        to_idx(1 - x, y, z),
        to_idx(x, y ^ 1, z),
        to_idx(x, y, z ^ 1),
        to_idx(x, y ^ 2, z),
        to_idx(x, y, z ^ 2),
    ]


def kernel(x, W1, W2):
    m, k = x.shape
    _, h = W1.shape
    _, n = W2.shape

    def body(x_ref, w1_ref, w2_ref, out_ref, acc_ref, recv_ref,
             send_sems, recv_sems):
        my = lax.axis_index("i")
        partners = _partners(my)

        xb = x_ref[...].astype(jnp.bfloat16)
        w1b = w1_ref[...].astype(jnp.bfloat16)
        w2b = w2_ref[...].astype(jnp.bfloat16)
        hid = jnp.dot(xb, w1b, preferred_element_type=jnp.float32)
        hid = jnp.maximum(hid, 0.0).astype(jnp.bfloat16)
        acc_ref[...] = jnp.dot(
            hid, w2b, preferred_element_type=jnp.float32
        ).astype(jnp.bfloat16)

        barrier_sem = pltpu.get_barrier_semaphore()
        for s in range(N_STEPS):
            pl.semaphore_signal(
                barrier_sem, inc=1,
                device_id=(partners[s],),
                device_id_type=pl.DeviceIdType.MESH,
            )
        pl.semaphore_wait(barrier_sem, N_STEPS)

        for s in range(N_STEPS):
            rdma = pltpu.make_async_remote_copy(
                src_ref=acc_ref,
                dst_ref=recv_ref.at[s],
                send_sem=send_sems.at[s],
                recv_sem=recv_sems.at[s],
                device_id=(partners[s],),
                device_id_type=pl.DeviceIdType.MESH,
            )
            rdma.start()
            rdma.wait()
            acc_ref[...] = acc_ref[...] + recv_ref[s]

        out_ref[...] = acc_ref[...].astype(jnp.float32)

    return pl.pallas_call(
        body,
        out_shape=jax.ShapeDtypeStruct((m, n), jnp.float32),
        in_specs=[pl.BlockSpec(memory_space=pltpu.VMEM)] * 3,
        out_specs=pl.BlockSpec(memory_space=pltpu.VMEM),
        scratch_shapes=[
            pltpu.VMEM((m, n), jnp.bfloat16),
            pltpu.VMEM((N_STEPS, m, n), jnp.bfloat16),
            pltpu.SemaphoreType.DMA((N_STEPS,)),
            pltpu.SemaphoreType.DMA((N_STEPS,)),
        ],
        compiler_params=pltpu.CompilerParams(collective_id=0),
    )(x, W1, W2)


# device time: 23455 ns/iter; 1.2707x vs baseline; 1.2089x over previous
import jax
import jax.numpy as jnp
from jax import lax
from jax.experimental import pallas as pl
from jax.experimental.pallas import tpu as pltpu

N_DEV = 32
N_STEPS = 5


def _partners(my):
    z = my // 8
    p = my % 8
    y = p // 2
    x = (p + y) % 2

    def to_idx(xx, yy, zz):
        return 8 * zz + 2 * yy + (xx + yy) % 2

    return [
        to_idx(1 - x, y, z),
        to_idx(x, y ^ 1, z),
        to_idx(x, y, z ^ 1),
        to_idx(x, y ^ 2, z),
        to_idx(x, y, z ^ 2),
    ]


def kernel(x, W1, W2):
    m, k = x.shape
    _, h = W1.shape
    _, n = W2.shape
    nh = n // 2

    def body(x_ref, w1_ref, w2_ref, out_ref,
             acc_a, acc_b, recv_a, recv_b,
             send_sems_a, recv_sems_a, send_sems_b, recv_sems_b):
        my = lax.axis_index("i")
        partners = _partners(my)
        order_a = [0, 1, 2, 3, 4]
        order_b = [1, 2, 3, 4, 0]

        xb = x_ref[...].astype(jnp.bfloat16)
        w1b = w1_ref[...].astype(jnp.bfloat16)
        w2b = w2_ref[...].astype(jnp.bfloat16)
        hid = jnp.dot(xb, w1b, preferred_element_type=jnp.float32)
        hid = jnp.maximum(hid, 0.0).astype(jnp.bfloat16)
        acc_a[...] = jnp.dot(
            hid, w2b[:, :nh], preferred_element_type=jnp.float32
        ).astype(jnp.bfloat16)

        barrier_sem = pltpu.get_barrier_semaphore()
        for s in range(N_STEPS):
            pl.semaphore_signal(
                barrier_sem, inc=1,
                device_id=(partners[s],),
                device_id_type=pl.DeviceIdType.MESH,
            )
        pl.semaphore_wait(barrier_sem, N_STEPS)

        def make(chain, t):
            acc, recv, ssems, rsems, order = (
                (acc_a, recv_a, send_sems_a, recv_sems_a, order_a)
                if chain == 0
                else (acc_b, recv_b, send_sems_b, recv_sems_b, order_b)
            )
            return pltpu.make_async_remote_copy(
                src_ref=acc,
                dst_ref=recv.at[t],
                send_sem=ssems.at[t],
                recv_sem=rsems.at[t],
                device_id=(partners[order[t]],),
                device_id_type=pl.DeviceIdType.MESH,
            )

        rdma_a = make(0, 0)
        rdma_a.start()
        acc_b[...] = jnp.dot(
            hid, w2b[:, nh:], preferred_element_type=jnp.float32
        ).astype(jnp.bfloat16)
        rdma_b = make(1, 0)
        rdma_b.start()

        for t in range(N_STEPS):
            rdma_a.wait()
            acc_a[...] = acc_a[...] + recv_a[t]
            if t + 1 < N_STEPS:
                rdma_a = make(0, t + 1)
                rdma_a.start()
            rdma_b.wait()
            acc_b[...] = acc_b[...] + recv_b[t]
            if t + 1 < N_STEPS:
                rdma_b = make(1, t + 1)
                rdma_b.start()

        out_ref[:, :nh] = acc_a[...].astype(jnp.float32)
        out_ref[:, nh:] = acc_b[...].astype(jnp.float32)

    return pl.pallas_call(
        body,
        out_shape=jax.ShapeDtypeStruct((m, n), jnp.float32),
        in_specs=[pl.BlockSpec(memory_space=pltpu.VMEM)] * 3,
        out_specs=pl.BlockSpec(memory_space=pltpu.VMEM),
        scratch_shapes=[
            pltpu.VMEM((m, nh), jnp.bfloat16),
            pltpu.VMEM((m, nh), jnp.bfloat16),
            pltpu.VMEM((N_STEPS, m, nh), jnp.bfloat16),
            pltpu.VMEM((N_STEPS, m, nh), jnp.bfloat16),
            pltpu.SemaphoreType.DMA((N_STEPS,)),
            pltpu.SemaphoreType.DMA((N_STEPS,)),
            pltpu.SemaphoreType.DMA((N_STEPS,)),
            pltpu.SemaphoreType.DMA((N_STEPS,)),
        ],
        compiler_params=pltpu.CompilerParams(collective_id=0),
    )(x, W1, W2)
